# baseline (speedup 1.0000x reference)
import jax
import jax.numpy as jnp
import numpy as np
from jax.experimental import pallas as pl
from jax.experimental.pallas import tpu as pltpu

_BLOCK = 32
_NLOCAL = 2
_STRIDE = 8
_TQ = 128
_SUPER = _STRIDE * _BLOCK
_LOCW = _TQ + 2 * _BLOCK


def _attn_kernel(q_ref, k_ref, v_ref, o_ref, ks_ref, vs_ref):
    t = pl.program_id(1)
    S = k_ref.shape[1]
    E = q_ref.shape[2]
    temp = 1.0 / float(np.sqrt(E))
    n_super = S // _SUPER

    @pl.when(t == 0)
    def _gather():
        for i in range(n_super):
            ks_ref[i * _BLOCK:(i + 1) * _BLOCK, :] = \
                k_ref[0, i * _SUPER:i * _SUPER + _BLOCK, :]
            vs_ref[i * _BLOCK:(i + 1) * _BLOCK, :] = \
                v_ref[0, i * _SUPER:i * _SUPER + _BLOCK, :]

    q = q_ref[0]
    ks = ks_ref[...]
    vs = vs_ref[...]

    start = jnp.clip(t * _TQ - _BLOCK, 0, S - _LOCW)
    kl = k_ref[0, pl.ds(start, _LOCW), :]
    vl = v_ref[0, pl.ds(start, _LOCW), :]

    dn = (((1,), (1,)), ((), ()))
    ss = jax.lax.dot_general(q, ks, dn,
                             preferred_element_type=jnp.float32) * temp
    sl = jax.lax.dot_general(q, kl, dn,
                             preferred_element_type=jnp.float32) * temp

    ns = ss.shape[1]
    bi_s = (jax.lax.broadcasted_iota(jnp.int32, (_TQ, ns), 0) + t * _TQ) // _BLOCK
    js = (jax.lax.broadcasted_iota(jnp.int32, (_TQ, ns), 1) // _BLOCK) * _STRIDE
    ss = jnp.where(jnp.abs(bi_s - js) >= _NLOCAL, ss, -1e30)

    bi_l = (jax.lax.broadcasted_iota(jnp.int32, (_TQ, _LOCW), 0) + t * _TQ) // _BLOCK
    jl = start // _BLOCK + jax.lax.broadcasted_iota(jnp.int32, (_TQ, _LOCW), 1) // _BLOCK
    sl = jnp.where(jnp.abs(bi_l - jl) < _NLOCAL, sl, -1e30)

    m = jnp.maximum(jnp.max(ss, axis=1), jnp.max(sl, axis=1))
    ps = jnp.exp(ss - m[:, None])
    plc = jnp.exp(sl - m[:, None])
    denom = jnp.sum(ps, axis=1) + jnp.sum(plc, axis=1)

    dv = (((1,), (0,)), ((), ()))
    out = jax.lax.dot_general(ps, vs, dv, preferred_element_type=jnp.float32)
    out = out + jax.lax.dot_general(plc, vl, dv,
                                    preferred_element_type=jnp.float32)
    o_ref[0] = out / denom[:, None]


def kernel(query, key, value):
    B, T, H, E = query.shape
    S = key.shape[1]
    q = jnp.transpose(query[0], (1, 0, 2))
    k = jnp.transpose(key[0], (1, 0, 2))
    v = jnp.transpose(value[0], (1, 0, 2))
    ns = (S // _SUPER) * _BLOCK

    grid = (H, T // _TQ)
    out = pl.pallas_call(
        _attn_kernel,
        grid=grid,
        in_specs=[
            pl.BlockSpec((1, _TQ, E), lambda h, t: (h, t, 0)),
            pl.BlockSpec((1, S, E), lambda h, t: (h, 0, 0)),
            pl.BlockSpec((1, S, E), lambda h, t: (h, 0, 0)),
        ],
        out_specs=pl.BlockSpec((1, _TQ, E), lambda h, t: (h, t, 0)),
        out_shape=jax.ShapeDtypeStruct((H, T, E), jnp.float32),
        scratch_shapes=[
            pltpu.VMEM((ns, E), jnp.float32),
            pltpu.VMEM((ns, E), jnp.float32),
        ],
        compiler_params=pltpu.CompilerParams(
            dimension_semantics=("parallel", "arbitrary"),
        ),
    )(q, k, v)
    return jnp.transpose(out, (1, 0, 2))[None]


# speedup vs baseline: 1.4227x; 1.4227x over previous
import jax
import jax.numpy as jnp
import numpy as np
from jax.experimental import pallas as pl
from jax.experimental.pallas import tpu as pltpu

_BLOCK = 32
_NLOCAL = 2
_STRIDE = 8
_TQ = 256
_SUPER = _STRIDE * _BLOCK
_LOCW = _TQ + 2 * _BLOCK


def _attn_kernel(q_ref, k_ref, v_ref, o_ref, ks_ref, vs_ref):
    t = pl.program_id(1)
    S = k_ref.shape[1]
    E = q_ref.shape[2]
    temp = 1.0 / float(np.sqrt(E))
    n_super = S // _SUPER

    @pl.when(t == 0)
    def _gather():
        for i in range(n_super):
            ks_ref[i * _BLOCK:(i + 1) * _BLOCK, :] = \
                k_ref[0, i * _SUPER:i * _SUPER + _BLOCK, :]
            vs_ref[i * _BLOCK:(i + 1) * _BLOCK, :] = \
                v_ref[0, i * _SUPER:i * _SUPER + _BLOCK, :]

    q = q_ref[0]
    ks = ks_ref[...]
    vs = vs_ref[...]

    start = jnp.clip(t * _TQ - _BLOCK, 0, S - _LOCW)
    kl = k_ref[0, pl.ds(start, _LOCW), :]
    vl = v_ref[0, pl.ds(start, _LOCW), :]

    dn = (((1,), (1,)), ((), ()))
    ss = jax.lax.dot_general(q, ks, dn,
                             preferred_element_type=jnp.float32) * temp
    sl = jax.lax.dot_general(q, kl, dn,
                             preferred_element_type=jnp.float32) * temp

    ns = ss.shape[1]
    bi_s = (jax.lax.broadcasted_iota(jnp.int32, (_TQ, ns), 0) + t * _TQ) // _BLOCK
    js = (jax.lax.broadcasted_iota(jnp.int32, (_TQ, ns), 1) // _BLOCK) * _STRIDE
    ss = jnp.where(jnp.abs(bi_s - js) >= _NLOCAL, ss, -1e30)

    bi_l = (jax.lax.broadcasted_iota(jnp.int32, (_TQ, _LOCW), 0) + t * _TQ) // _BLOCK
    jl = start // _BLOCK + jax.lax.broadcasted_iota(jnp.int32, (_TQ, _LOCW), 1) // _BLOCK
    sl = jnp.where(jnp.abs(bi_l - jl) < _NLOCAL, sl, -1e30)

    m = jnp.maximum(jnp.max(ss, axis=1), jnp.max(sl, axis=1))
    ps = jnp.exp(ss - m[:, None])
    plc = jnp.exp(sl - m[:, None])
    denom = jnp.sum(ps, axis=1) + jnp.sum(plc, axis=1)

    dv = (((1,), (0,)), ((), ()))
    out = jax.lax.dot_general(ps, vs, dv, preferred_element_type=jnp.float32)
    out = out + jax.lax.dot_general(plc, vl, dv,
                                    preferred_element_type=jnp.float32)
    o_ref[0] = out / denom[:, None]


def kernel(query, key, value):
    B, T, H, E = query.shape
    S = key.shape[1]
    q = jnp.transpose(query[0], (1, 0, 2))
    k = jnp.transpose(key[0], (1, 0, 2))
    v = jnp.transpose(value[0], (1, 0, 2))
    ns = (S // _SUPER) * _BLOCK

    grid = (H, T // _TQ)
    out = pl.pallas_call(
        _attn_kernel,
        grid=grid,
        in_specs=[
            pl.BlockSpec((1, _TQ, E), lambda h, t: (h, t, 0)),
            pl.BlockSpec((1, S, E), lambda h, t: (h, 0, 0)),
            pl.BlockSpec((1, S, E), lambda h, t: (h, 0, 0)),
        ],
        out_specs=pl.BlockSpec((1, _TQ, E), lambda h, t: (h, t, 0)),
        out_shape=jax.ShapeDtypeStruct((H, T, E), jnp.float32),
        scratch_shapes=[
            pltpu.VMEM((ns, E), jnp.float32),
            pltpu.VMEM((ns, E), jnp.float32),
        ],
        compiler_params=pltpu.CompilerParams(
            dimension_semantics=("parallel", "arbitrary"),
        ),
    )(q, k, v)
    return jnp.transpose(out, (1, 0, 2))[None]


# speedup vs baseline: 2.1062x; 1.4804x over previous
import jax
import jax.numpy as jnp
import numpy as np
from jax.experimental import pallas as pl
from jax.experimental.pallas import tpu as pltpu

_BLOCK = 32
_NLOCAL = 2
_STRIDE = 8
_TQ = 256
_SUPER = _STRIDE * _BLOCK
_LOCW = _TQ + 2 * _BLOCK
_NEG = -1e30


def _local_start(t, S):
    return min(max(t * _TQ - _BLOCK, 0), S - _LOCW)


def _make_biases(T, S):
    ns = (S // _SUPER) * _BLOCK
    rows = np.arange(T)[:, None] // _BLOCK
    cs = np.arange(ns)[None, :] // _BLOCK * _STRIDE
    bias_s = np.where(np.abs(rows - cs) >= _NLOCAL, 0.0, _NEG).astype(np.float32)

    bias_l = np.full((T, _LOCW), _NEG, dtype=np.float32)
    for t in range(T // _TQ):
        start = _local_start(t, S)
        r = np.arange(t * _TQ, (t + 1) * _TQ)[:, None] // _BLOCK
        c = start // _BLOCK + np.arange(_LOCW)[None, :] // _BLOCK
        bias_l[t * _TQ:(t + 1) * _TQ] = np.where(
            np.abs(r - c) < _NLOCAL, 0.0, _NEG)
    return bias_s, bias_l


def _attn_kernel(q_ref, k_ref, v_ref, bs_ref, bl_ref, o_ref, ks_ref, vs_ref):
    S = k_ref.shape[1]
    n_super = S // _SUPER

    for i in range(n_super):
        ks_ref[i * _BLOCK:(i + 1) * _BLOCK, :] = \
            k_ref[0, i * _SUPER:i * _SUPER + _BLOCK, :]
        vs_ref[i * _BLOCK:(i + 1) * _BLOCK, :] = \
            v_ref[0, i * _SUPER:i * _SUPER + _BLOCK, :]
    ks = ks_ref[...]
    vs = vs_ref[...]

    dn = (((1,), (1,)), ((), ()))
    dv = (((1,), (0,)), ((), ()))
    for t in range(q_ref.shape[1] // _TQ):
        q = q_ref[0, t * _TQ:(t + 1) * _TQ, :]
        start = _local_start(t, S)
        kl = k_ref[0, start:start + _LOCW, :]
        vl = v_ref[0, start:start + _LOCW, :]

        ss = jax.lax.dot_general(q, ks, dn, preferred_element_type=jnp.float32)
        ss = ss + bs_ref[t * _TQ:(t + 1) * _TQ, :]
        sl = jax.lax.dot_general(q, kl, dn, preferred_element_type=jnp.float32)
        sl = sl + bl_ref[t * _TQ:(t + 1) * _TQ, :]

        m = jnp.maximum(jnp.max(ss, axis=1), jnp.max(sl, axis=1))
        ps = jnp.exp(ss - m[:, None])
        plc = jnp.exp(sl - m[:, None])
        denom = jnp.sum(ps, axis=1) + jnp.sum(plc, axis=1)

        out = jax.lax.dot_general(ps, vs, dv, preferred_element_type=jnp.float32)
        out = out + jax.lax.dot_general(plc, vl, dv,
                                        preferred_element_type=jnp.float32)
        o_ref[0, t * _TQ:(t + 1) * _TQ, :] = out / denom[:, None]


def kernel(query, key, value):
    B, T, H, E = query.shape
    S = key.shape[1]
    temp = 1.0 / float(np.sqrt(E))
    q = jnp.transpose(query[0], (1, 0, 2)) * temp
    k = jnp.transpose(key[0], (1, 0, 2))
    v = jnp.transpose(value[0], (1, 0, 2))
    ns = (S // _SUPER) * _BLOCK
    bias_s, bias_l = _make_biases(T, S)

    out = pl.pallas_call(
        _attn_kernel,
        grid=(H,),
        in_specs=[
            pl.BlockSpec((1, T, E), lambda h: (h, 0, 0)),
            pl.BlockSpec((1, S, E), lambda h: (h, 0, 0)),
            pl.BlockSpec((1, S, E), lambda h: (h, 0, 0)),
            pl.BlockSpec((T, ns), lambda h: (0, 0)),
            pl.BlockSpec((T, _LOCW), lambda h: (0, 0)),
        ],
        out_specs=pl.BlockSpec((1, _TQ * (T // _TQ), E), lambda h: (h, 0, 0)),
        out_shape=jax.ShapeDtypeStruct((H, T, E), jnp.float32),
        scratch_shapes=[
            pltpu.VMEM((ns, E), jnp.float32),
            pltpu.VMEM((ns, E), jnp.float32),
        ],
        compiler_params=pltpu.CompilerParams(
            dimension_semantics=("parallel",),
        ),
    )(q, k, v, jnp.asarray(bias_s), jnp.asarray(bias_l))
    return jnp.transpose(out, (1, 0, 2))[None]
